# baseline (speedup 1.0000x reference)
import dataclasses
import functools

import jax
import jax.numpy as jnp
from jax import lax
from jax.experimental import pallas as pl
from jax.experimental.pallas import tpu as pltpu
from jax.experimental.pallas import tpu_sc as plsc

COMMITMENT = 0.25
SC_LANES = 16
NUM_WORKERS = 32


def _vq_tc_body(z_ref, e_ref, idx_ref, loss_ref):
    C = z_ref.shape[1]
    zb = z_ref[0].reshape(C, -1)
    emb = e_ref[...]
    K = emb.shape[0]
    N = zb.shape[1]
    nm2 = -(emb + emb)
    scores2 = lax.dot_general(
        nm2, zb, (((1,), (0,)), ((), ())),
        preferred_element_type=jnp.float32)
    esq = jnp.sum(emb * emb, axis=1, keepdims=True)
    t = zb * zb
    while t.shape[0] > 1:
        h = t.shape[0] // 2
        t = t[:h] + t[h:]
    zsq = t
    dist = (zsq + scores2) + esq
    m = jnp.min(dist, axis=0, keepdims=True)
    kiota = lax.broadcasted_iota(jnp.int32, (K, 1), 0).astype(jnp.float32)
    idxf = jnp.min(jnp.where(dist == m, kiota, float(K)), axis=0)
    idx_ref[0, 0, :] = idxf.astype(jnp.int32)
    loss_ref[0, 0, :] = jnp.broadcast_to(jnp.sum(m), (loss_ref.shape[2],))


def _sc_gather_body(chunk, e_hbm, idx_hbm, out_hbm, idxv, rows_v, sem):
    wid = lax.axis_index("s") * 2 + lax.axis_index("c")
    B, H, W, D = out_hbm.shape
    nq = (H * W) // chunk
    b = wid // nq
    h0 = (wid % nq) * (chunk // W)
    pltpu.sync_copy(idx_hbm.at[b, 0, pl.ds(h0 * W, chunk)], idxv)
    copies = []
    for k in range(chunk // 128):
        copies.append(pltpu.async_copy(
            e_hbm.at[idxv.at[pl.ds(k * 128, 128)]],
            rows_v.at[pl.ds(k * 128, 128)], sem))
    for cp in copies:
        cp.wait()
    stores = []
    for g in range(chunk // W):
        stores.append(pltpu.async_copy(
            rows_v.at[pl.ds(g * W, W)], out_hbm.at[b, h0 + g], sem))
    for st in stores:
        st.wait()


def kernel(z, embedding_weight):
    B, C, H, W = z.shape
    K, D = embedding_weight.shape
    N = H * W

    idx3, loss3 = pl.pallas_call(
        _vq_tc_body,
        grid=(B,),
        in_specs=[
            pl.BlockSpec((1, C, H, W), lambda b: (b, 0, 0, 0)),
            pl.BlockSpec((K, D), lambda b: (0, 0)),
        ],
        out_specs=[
            pl.BlockSpec((1, 1, N), lambda b: (b, 0, 0)),
            pl.BlockSpec((1, 1, 128), lambda b: (b, 0, 0)),
        ],
        out_shape=[
            jax.ShapeDtypeStruct((B, 1, N), jnp.int32),
            jax.ShapeDtypeStruct((B, 1, 128), jnp.float32),
        ],
    )(z, embedding_weight)

    chunk = (B * N) // NUM_WORKERS
    sc_params = pltpu.CompilerParams(
        needs_layout_passes=False, use_tc_tiling_on_sc=False)
    sc_gather = pl.kernel(
        functools.partial(_sc_gather_body, chunk),
        out_type=jax.ShapeDtypeStruct((B, H, W, D), jnp.float32),
        mesh=plsc.VectorSubcoreMesh(core_axis_name="c", subcore_axis_name="s"),
        scratch_types=[
            pltpu.VMEM((chunk,), jnp.int32),
            pltpu.VMEM((chunk, D), jnp.float32),
            pltpu.SemaphoreType.DMA,
        ],
        compiler_params=sc_params,
    )
    rows4 = sc_gather(embedding_weight, idx3)

    quant = jnp.transpose(rows4, (0, 3, 1, 2))
    s = loss3[:, 0, 0]
    vq_loss = jnp.mean(s + COMMITMENT * s)
    return (quant, vq_loss, idx3.reshape(B, N))


# speedup vs baseline: 1.0871x; 1.0871x over previous
import dataclasses
import functools

import jax
import jax.numpy as jnp
from jax import lax
from jax.experimental import pallas as pl
from jax.experimental.pallas import tpu as pltpu
from jax.experimental.pallas import tpu_sc as plsc

COMMITMENT = 0.25
SC_LANES = 16
NUM_WORKERS = 32


def _vq_tc_body(z_ref, e_ref, idx_ref, loss_ref):
    C = z_ref.shape[1]
    zb = z_ref[0].reshape(C, -1)
    emb = e_ref[...]
    K = emb.shape[0]
    N = zb.shape[1]
    nm2 = -(emb + emb)
    scores2 = lax.dot_general(
        nm2, zb, (((1,), (0,)), ((), ())),
        preferred_element_type=jnp.float32)
    esq = jnp.sum(emb * emb, axis=1, keepdims=True)
    t = zb * zb
    while t.shape[0] > 1:
        h = t.shape[0] // 2
        t = t[:h] + t[h:]
    zsq = t
    dist = (zsq + scores2) + esq
    m = jnp.min(dist, axis=0, keepdims=True)
    kiota = lax.broadcasted_iota(jnp.int32, (K, 1), 0).astype(jnp.float32)
    idxf = jnp.min(jnp.where(dist == m, kiota, float(K)), axis=0)
    idx_ref[0, 0, :] = idxf.astype(jnp.int32)
    loss_ref[0, 0, :] = jnp.broadcast_to(jnp.sum(m), (loss_ref.shape[2],))


def _sc_gather_body(chunk, e_hbm, idx_hbm, out_hbm, idxv, rows_v, sem):
    wid = lax.axis_index("s") * 2 + lax.axis_index("c")
    B, N, D = out_hbm.shape
    nq = N // chunk
    b = wid // nq
    n0 = (wid % nq) * chunk
    pltpu.sync_copy(idx_hbm.at[b, 0, pl.ds(n0, chunk)], idxv)
    copies = []
    for k in range(chunk // 128):
        copies.append(pltpu.async_copy(
            e_hbm.at[idxv.at[pl.ds(k * 128, 128)]],
            rows_v.at[pl.ds(k * 128, 128)], sem))
    for cp in copies:
        cp.wait()
    pltpu.sync_copy(rows_v, out_hbm.at[b, pl.ds(n0, chunk)])


def _fin_body(x_ref, o_ref):
    x = x_ref[0]
    C, H, W = o_ref.shape[1:]
    parts = [jnp.transpose(x[:, q * C:(q + 1) * C]) for q in range(4)]
    o_ref[0] = jnp.concatenate(parts, axis=1).reshape(C, H, W)


def kernel(z, embedding_weight):
    B, C, H, W = z.shape
    K, D = embedding_weight.shape
    N = H * W

    idx3, loss3 = pl.pallas_call(
        _vq_tc_body,
        grid=(B,),
        in_specs=[
            pl.BlockSpec((1, C, H, W), lambda b: (b, 0, 0, 0)),
            pl.BlockSpec((K, D), lambda b: (0, 0)),
        ],
        out_specs=[
            pl.BlockSpec((1, 1, N), lambda b: (b, 0, 0)),
            pl.BlockSpec((1, 1, 128), lambda b: (b, 0, 0)),
        ],
        out_shape=[
            jax.ShapeDtypeStruct((B, 1, N), jnp.int32),
            jax.ShapeDtypeStruct((B, 1, 128), jnp.float32),
        ],
    )(z, embedding_weight)

    chunk = (B * N) // NUM_WORKERS
    sc_params = pltpu.CompilerParams(
        needs_layout_passes=False, use_tc_tiling_on_sc=False)
    sc_gather = pl.kernel(
        functools.partial(_sc_gather_body, chunk),
        out_type=jax.ShapeDtypeStruct((B, N, D), jnp.float32),
        mesh=plsc.VectorSubcoreMesh(core_axis_name="c", subcore_axis_name="s"),
        scratch_types=[
            pltpu.VMEM((chunk,), jnp.int32),
            pltpu.VMEM((chunk, D), jnp.float32),
            pltpu.SemaphoreType.DMA,
        ],
        compiler_params=sc_params,
    )
    idx_pm = idx3.reshape(B, 4, N // 4).transpose(0, 2, 1).reshape(B, 1, N)
    rows = sc_gather(embedding_weight, idx_pm)

    quant = pl.pallas_call(
        _fin_body,
        grid=(B,),
        in_specs=[pl.BlockSpec((1, N * D // 128, 128), lambda b: (b, 0, 0))],
        out_specs=pl.BlockSpec((1, C, H, W), lambda b: (b, 0, 0, 0)),
        out_shape=jax.ShapeDtypeStruct((B, C, H, W), jnp.float32),
    )(rows.reshape(B, N * D // 128, 128))

    s = loss3[:, 0, 0]
    vq_loss = jnp.mean(s + COMMITMENT * s)
    return (quant, vq_loss, idx3.reshape(B, N))
